# baseline (speedup 1.0000x reference)
import jax
import jax.numpy as jnp
from jax.experimental import pallas as pl
from jax.experimental.pallas import tpu as pltpu

_TARGET_TILE = 2048


def _pick_tile(hw: int) -> int:
    if hw % 128 != 0:
        return hw
    best = 128
    t = 128
    while t <= min(hw, _TARGET_TILE):
        if hw % t == 0:
            best = t
        t += 128
    return best


def _train_kernel(x_ref, wt_ref, b_ref, lab_ref,
                  plab_ref, lab_part_ref, unlab_part_ref):
    z = jnp.dot(wt_ref[...], x_ref[0],
                preferred_element_type=jnp.float32) + b_ref[...]
    m = jnp.max(z, axis=0, keepdims=True)
    plab = jnp.argmax(z, axis=0)
    plab_ref[0] = plab.reshape(1, -1).astype(jnp.int32)
    s = jnp.sum(jnp.exp(z - m), axis=0, keepdims=True)
    logs = jnp.log(s)
    cls_iota = jax.lax.broadcasted_iota(jnp.int32, z.shape, 0)
    picked = jnp.sum(jnp.where(cls_iota == lab_ref[0], z, 0.0),
                     axis=0, keepdims=True)
    lab_sum = jnp.sum(m + logs - picked)
    unlab_sum = jnp.sum(logs)
    lab_part_ref[...] = jnp.full(lab_part_ref.shape, lab_sum, jnp.float32)
    unlab_part_ref[...] = jnp.full(unlab_part_ref.shape, unlab_sum,
                                   jnp.float32)


def kernel(x, weight, bias, labels):
    B, Cin, H, W = x.shape
    C = weight.shape[1]
    HW = H * W
    T = _pick_tile(HW)
    nt = HW // T

    x_chw = x.reshape(B, Cin, HW)
    w_t = weight.T
    b_col = bias.reshape(C, 1)
    labels3 = labels.reshape(B, 1, HW).astype(jnp.int32)

    plab3, lab_part, unlab_part = pl.pallas_call(
        _train_kernel,
        out_shape=(
            jax.ShapeDtypeStruct((B, 1, HW), jnp.int32),
            jax.ShapeDtypeStruct((B, nt, 1, 128), jnp.float32),
            jax.ShapeDtypeStruct((B, nt, 1, 128), jnp.float32),
        ),
        grid=(B, nt),
        in_specs=[
            pl.BlockSpec((1, Cin, T), lambda b, t: (b, 0, t)),
            pl.BlockSpec((C, Cin), lambda b, t: (0, 0)),
            pl.BlockSpec((C, 1), lambda b, t: (0, 0)),
            pl.BlockSpec((1, 1, T), lambda b, t: (b, 0, t)),
        ],
        out_specs=(
            pl.BlockSpec((1, 1, T), lambda b, t: (b, 0, t)),
            pl.BlockSpec((1, 1, 1, 128), lambda b, t: (b, t, 0, 0)),
            pl.BlockSpec((1, 1, 1, 128), lambda b, t: (b, t, 0, 0)),
        ),
        compiler_params=pltpu.CompilerParams(
            dimension_semantics=("parallel", "parallel"),
        ),
    )(x_chw, w_t, b_col, labels3)

    denom = B * HW
    return (plab3.reshape(B, HW),
            jnp.sum(lab_part[..., 0]) / denom,
            jnp.sum(unlab_part[..., 0]) / denom)
